# baseline (speedup 1.0000x reference)
import functools
import math

import jax
import jax.numpy as jnp
from jax import lax
from jax.experimental import pallas as pl
from jax.experimental.pallas import tpu as pltpu
from jax.experimental.pallas import tpu_sc as plsc

B, CH, H, W = 16, 85, 160, 160
HW = H * W
N, NC = 32, 80
NPTS = B * N
SLOTS = 48
OUTW = SLOTS * N

def _sc_gather_body(x3_hbm, cy_hbm, cx_hbm, out_hbm, cyv, cxv, qbuf, gvout,
                    semg):
    wid = lax.axis_index("s") * 2 + lax.axis_index("c")
    b = wid % 16
    c0 = (wid // 16) * SLOTS
    pltpu.sync_copy(cy_hbm.at[pl.ds(b * N, N)], cyv)
    pltpu.sync_copy(cx_hbm.at[pl.ds(b * N, N)], cxv)
    qbuf[pl.ds(0, 16)] = cyv[pl.ds(0, 16)] * W + cxv[pl.ds(0, 16)]
    qbuf[pl.ds(16, 16)] = cyv[pl.ds(16, 16)] * W + cxv[pl.ds(16, 16)]

    def chan(cs):
        c = c0 + cs
        return jnp.where(c > CH - 1, CH - 1, c)

    def fire(i, carry):
        pltpu.async_copy(x3_hbm.at[b, chan(i)].at[qbuf],
                         gvout.at[pl.ds(i * N, N)], semg)
        return carry

    lax.fori_loop(0, SLOTS, fire, 0)

    def drain(i, carry):
        pltpu.make_async_copy(x3_hbm.at[b, chan(i)].at[qbuf],
                              gvout.at[pl.ds(i * N, N)], semg).wait()
        return carry

    lax.fori_loop(0, SLOTS, drain, 0)
    pltpu.sync_copy(gvout, out_hbm.at[wid])


@functools.cache
def _sc_gather_kernel():
    mesh = plsc.VectorSubcoreMesh(core_axis_name="c", subcore_axis_name="s")
    return pl.kernel(
        _sc_gather_body,
        mesh=mesh,
        compiler_params=pltpu.CompilerParams(use_tc_tiling_on_sc=False),
        out_type=jax.ShapeDtypeStruct((32, OUTW), jnp.float32),
        scratch_types=[
            pltpu.VMEM((N,), jnp.int32),
            pltpu.VMEM((N,), jnp.int32),
            pltpu.VMEM((N,), jnp.int32),
            pltpu.VMEM((OUTW,), jnp.float32),
            pltpu.SemaphoreType.DMA,
        ],
    )


def _atan(x):
    a = jnp.abs(x)
    inv = a > 1.0
    t = jnp.where(inv, 1.0 / jnp.maximum(a, 1e-30), a)
    s = t * t
    p = t * (0.99997726 + s * (-0.33262347 + s * (0.19354346 + s * (
        -0.11643287 + s * (0.05265332 + s * (-0.01172120))))))
    p = jnp.where(inv, (math.pi / 2.0) - p, p)
    return jnp.where(x < 0.0, -p, p)


def _softplus(x):
    return jnp.maximum(x, 0.0) + jnp.log(1.0 + jnp.exp(-jnp.abs(x)))


def _tc_body(x_ref, g_ref, gtcf_ref, gtb_ref, cy_ref, cx_ref, cyc_ref,
             cxc_ref, out_ref):
    eps = 1e-10
    conf = x_ref[:, 0, :, :]
    neg_all = jnp.sum(_softplus(conf))

    cy32 = cy_ref[...]
    cx32 = cx_ref[...]
    bidx = lax.broadcasted_iota(jnp.int32, (B, N), 0)
    key = (bidx * H + cy32) * W + cx32
    bflat = lax.broadcasted_iota(jnp.int32, (1, NPTS), 1) // N
    keyflat = (bflat * H + cyc_ref[...]) * W + cxc_ref[...]
    eq3 = key[:, :, None] == keyflat.reshape(1, 1, NPTS)
    rowid = (bidx * N + lax.broadcasted_iota(jnp.int32, (B, N), 1))[:, :, None]
    colid = lax.broadcasted_iota(jnp.int32, (B, N, NPTS), 2)
    dup = jnp.any(eq3 & (colid < rowid), axis=2)
    uniq = jnp.where(dup, 0.0, 1.0)

    confc = g_ref[0:B, 0:N]
    sp_neg = _softplus(confc)
    sp_pos = sp_neg - confc
    pos_cnt = jnp.sum(uniq)
    conf_pos = jnp.sum(uniq * sp_pos) / jnp.maximum(pos_cnt, 1.0)
    conf_neg = (neg_all - jnp.sum(uniq * sp_neg)) / jnp.maximum(
        float(B * H * W) - pos_cnt, 1.0)

    l1, t1 = g_ref[0:B, N:2 * N], g_ref[0:B, 2 * N:3 * N]
    r1, b1 = g_ref[0:B, 3 * N:4 * N], g_ref[0:B, 4 * N:5 * N]
    l2, t2 = gtb_ref[:, 0, :], gtb_ref[:, 1, :]
    r2, b2 = gtb_ref[:, 2, :], gtb_ref[:, 3, :]
    w1, h1 = r1 - l1, b1 - t1
    w2, h2 = r2 - l2, b2 - t2
    inter = jnp.clip(jnp.minimum(r1, r2) - jnp.maximum(l1, l2), 0.0, None) * \
            jnp.clip(jnp.minimum(b1, b2) - jnp.maximum(t1, t2), 0.0, None)
    union = w1 * h1 + w2 * h2 - inter + eps
    iou = inter / union
    cw = jnp.maximum(r1, r2) - jnp.minimum(l1, l2)
    chh = jnp.maximum(b1, b2) - jnp.minimum(t1, t2)
    c2 = cw ** 2 + chh ** 2 + eps
    rho2 = ((l2 + r2 - l1 - r1) ** 2 + (b2 + t2 - b1 - t1) ** 2) / 4.0
    v = 4.0 / (math.pi ** 2) * (_atan(w2 / (h2 + eps)) - _atan(w1 / (h1 + eps))) ** 2
    alpha = v / (v - iou + (1.0 + eps))
    ciou = iou - (rho2 / c2 + v * alpha)
    bbox_loss = -jnp.sum(ciou) / float(NPTS)

    cls_loss = -jnp.sum(g_ref[...] * gtcf_ref[...]) / float(NPTS)

    total = conf_pos + conf_neg + bbox_loss + cls_loss
    out_ref[...] = jnp.reshape(total, (1, 1))


def _tc_loss(x, g2, gtcf, gtb3, cy32, cx32, cy_c, cx_c, interpret=False):
    return pl.pallas_call(
        _tc_body,
        grid=(1,),
        in_specs=[
            pl.BlockSpec((B, 1, H, W), lambda i: (0, 0, 0, 0)),
            pl.BlockSpec((32, OUTW), lambda i: (0, 0)),
            pl.BlockSpec((32, OUTW), lambda i: (0, 0)),
            pl.BlockSpec((B, 4, N), lambda i: (0, 0, 0)),
            pl.BlockSpec((B, N), lambda i: (0, 0)),
            pl.BlockSpec((B, N), lambda i: (0, 0)),
            pl.BlockSpec((1, NPTS), lambda i: (0, 0)),
            pl.BlockSpec((1, NPTS), lambda i: (0, 0)),
        ],
        out_specs=pl.BlockSpec((1, 1), lambda i: (0, 0)),
        out_shape=jax.ShapeDtypeStruct((1, 1), jnp.float32),
        interpret=interpret,
    )(x, g2, gtcf, gtb3, cy32, cx32, cy_c, cx_c)


def _build_gtcf(gt_cls):
    gtT = jnp.transpose(gt_cls, (0, 2, 1))
    z5 = jnp.zeros((B, 5, N), jnp.float32)
    a = jnp.concatenate([z5, gtT[:, :SLOTS - 5, :]], axis=1)
    z11 = jnp.zeros((B, SLOTS - (CH - SLOTS), N), jnp.float32)
    bm = jnp.concatenate([gtT[:, SLOTS - 5:, :], z11], axis=1)
    return jnp.concatenate([a, bm], axis=0).reshape(32, OUTW)


def kernel(x, gt_bbox, gt_cls, cy, cx):
    g2 = _sc_gather_kernel()(x.reshape(B, CH, HW), cy.reshape(-1),
                             cx.reshape(-1))
    out = _tc_loss(
        x, g2,
        _build_gtcf(gt_cls),
        jnp.transpose(gt_bbox, (0, 2, 1)),
        cy, cx,
        cy.reshape(1, NPTS), cx.reshape(1, NPTS),
    )
    return out[0, 0]


# speedup vs baseline: 1.8179x; 1.8179x over previous
import math

import jax
import jax.numpy as jnp
from jax import lax
from jax.experimental import pallas as pl
from jax.experimental.pallas import tpu as pltpu

B, CH, H, W = 16, 85, 160, 160
HW = H * W
N, NC = 32, 80
NPTS = B * N


def _tc_gather_body(cy_ref, cx_ref, x_ref, g_ref, slabs, sem):
    b = pl.program_id(0)
    copies = []
    for n in range(N):
        c = pltpu.make_async_copy(
            x_ref.at[b, :, cy_ref[b, n], :], slabs.at[n], sem)
        c.start()
        copies.append(c)
    for c in copies:
        c.wait()
    wio = lax.broadcasted_iota(jnp.int32, (1, W), 1)
    cols = []
    for n in range(N):
        mask = (wio == cx_ref[b, n]).astype(jnp.float32)
        cols.append(jnp.sum(slabs[n] * mask, axis=1, keepdims=True))
    g_ref[0] = jnp.concatenate(cols, axis=1)


def _tc_gather(x, cy, cx):
    return pl.pallas_call(
        _tc_gather_body,
        grid=(B,),
        in_specs=[
            pl.BlockSpec(memory_space=pltpu.SMEM),
            pl.BlockSpec(memory_space=pltpu.SMEM),
            pl.BlockSpec(memory_space=pltpu.MemorySpace.HBM),
        ],
        out_specs=pl.BlockSpec((1, CH, N), lambda b: (b, 0, 0)),
        out_shape=jax.ShapeDtypeStruct((B, CH, N), jnp.float32),
        scratch_shapes=[
            pltpu.VMEM((N, CH, W), jnp.float32),
            pltpu.SemaphoreType.DMA,
        ],
    )(cy, cx, x)


def _atan(x):
    a = jnp.abs(x)
    inv = a > 1.0
    t = jnp.where(inv, 1.0 / jnp.maximum(a, 1e-30), a)
    s = t * t
    p = t * (0.99997726 + s * (-0.33262347 + s * (0.19354346 + s * (
        -0.11643287 + s * (0.05265332 + s * (-0.01172120))))))
    p = jnp.where(inv, (math.pi / 2.0) - p, p)
    return jnp.where(x < 0.0, -p, p)


def _softplus(x):
    return jnp.maximum(x, 0.0) + jnp.log(1.0 + jnp.exp(-jnp.abs(x)))


def _tc_loss_body(x_ref, g_ref, gtt_ref, gtb_ref, cy_ref, cx_ref, cyc_ref,
                  cxc_ref, out_ref):
    eps = 1e-10
    conf = x_ref[:, 0, :, :]
    neg_all = jnp.sum(_softplus(conf))

    cy32 = cy_ref[...]
    cx32 = cx_ref[...]
    bidx = lax.broadcasted_iota(jnp.int32, (B, N), 0)
    key = (bidx * H + cy32) * W + cx32
    bflat = lax.broadcasted_iota(jnp.int32, (1, NPTS), 1) // N
    keyflat = (bflat * H + cyc_ref[...]) * W + cxc_ref[...]
    eq3 = key[:, :, None] == keyflat.reshape(1, 1, NPTS)
    rowid = (bidx * N + lax.broadcasted_iota(jnp.int32, (B, N), 1))[:, :, None]
    colid = lax.broadcasted_iota(jnp.int32, (B, N, NPTS), 2)
    dup = jnp.any(eq3 & (colid < rowid), axis=2)
    uniq = jnp.where(dup, 0.0, 1.0)

    confc = g_ref[:, 0, :]
    sp_neg = _softplus(confc)
    sp_pos = sp_neg - confc
    pos_cnt = jnp.sum(uniq)
    conf_pos = jnp.sum(uniq * sp_pos) / jnp.maximum(pos_cnt, 1.0)
    conf_neg = (neg_all - jnp.sum(uniq * sp_neg)) / jnp.maximum(
        float(B * H * W) - pos_cnt, 1.0)

    l1, t1, r1, b1 = (g_ref[:, 1, :], g_ref[:, 2, :],
                      g_ref[:, 3, :], g_ref[:, 4, :])
    l2, t2 = gtb_ref[:, 0, :], gtb_ref[:, 1, :]
    r2, b2 = gtb_ref[:, 2, :], gtb_ref[:, 3, :]
    w1, h1 = r1 - l1, b1 - t1
    w2, h2 = r2 - l2, b2 - t2
    inter = jnp.clip(jnp.minimum(r1, r2) - jnp.maximum(l1, l2), 0.0, None) * \
            jnp.clip(jnp.minimum(b1, b2) - jnp.maximum(t1, t2), 0.0, None)
    union = w1 * h1 + w2 * h2 - inter + eps
    iou = inter / union
    cw = jnp.maximum(r1, r2) - jnp.minimum(l1, l2)
    chh = jnp.maximum(b1, b2) - jnp.minimum(t1, t2)
    c2 = cw ** 2 + chh ** 2 + eps
    rho2 = ((l2 + r2 - l1 - r1) ** 2 + (b2 + t2 - b1 - t1) ** 2) / 4.0
    v = 4.0 / (math.pi ** 2) * (_atan(w2 / (h2 + eps)) - _atan(w1 / (h1 + eps))) ** 2
    alpha = v / (v - iou + (1.0 + eps))
    ciou = iou - (rho2 / c2 + v * alpha)
    bbox_loss = -jnp.sum(ciou) / float(NPTS)

    cls_loss = -jnp.sum(g_ref[:, 5:, :] * gtt_ref[...]) / float(NPTS)

    total = conf_pos + conf_neg + bbox_loss + cls_loss
    out_ref[...] = jnp.reshape(total, (1, 1))


def _tc_loss(x, g3, gtt, gtb3, cy32, cx32, cy_c, cx_c, interpret=False):
    return pl.pallas_call(
        _tc_loss_body,
        grid=(1,),
        in_specs=[
            pl.BlockSpec((B, 1, H, W), lambda i: (0, 0, 0, 0)),
            pl.BlockSpec((B, CH, N), lambda i: (0, 0, 0)),
            pl.BlockSpec((B, NC, N), lambda i: (0, 0, 0)),
            pl.BlockSpec((B, 4, N), lambda i: (0, 0, 0)),
            pl.BlockSpec((B, N), lambda i: (0, 0)),
            pl.BlockSpec((B, N), lambda i: (0, 0)),
            pl.BlockSpec((1, NPTS), lambda i: (0, 0)),
            pl.BlockSpec((1, NPTS), lambda i: (0, 0)),
        ],
        out_specs=pl.BlockSpec((1, 1), lambda i: (0, 0)),
        out_shape=jax.ShapeDtypeStruct((1, 1), jnp.float32),
        interpret=interpret,
    )(x, g3, gtt, gtb3, cy32, cx32, cy_c, cx_c)


def kernel(x, gt_bbox, gt_cls, cy, cx):
    g3 = _tc_gather(x, cy, cx)
    out = _tc_loss(
        x, g3,
        jnp.transpose(gt_cls, (0, 2, 1)),
        jnp.transpose(gt_bbox, (0, 2, 1)),
        cy, cx,
        cy.reshape(1, NPTS), cx.reshape(1, NPTS),
    )
    return out[0, 0]
